# Initial kernel scaffold; baseline (speedup 1.0000x reference)
import jax
import jax.numpy as jnp
from jax.experimental import pallas as pl


def kernel(atom_feature, edge_index, bond_feature, node2graph, W_in, b_in, W_msg, b_msg, W_upd, b_upd, W_out, b_out):
    raise NotImplementedError("write your pallas kernel here")


# speedup vs baseline: 2.9276x; 2.9276x over previous
import functools

import jax
import jax.numpy as jnp
from jax import lax
from jax.experimental import pallas as pl
from jax.experimental.pallas import tpu as pltpu
from jax.experimental.pallas import tpu_sc as plsc

_N, _E, _DIN, _H, _DE, _NBLK = 10000, 320000, 1422, 128, 8, 5
_NC, _NS = 2, 16
_NW = _NC * _NS
_CH = 128
_NCHUNK = 79
_EPW = _NCHUNK * _CH
_EPAD = _NW * _EPW
_NPAD = 10240
_RPT = _NPAD // _NS
_DINP = 1424


def _proj_body(a_ref, w_ref, b_ref, o_ref):
    o_ref[...] = jnp.maximum(
        jnp.dot(a_ref[...], w_ref[...], preferred_element_type=jnp.float32)
        + b_ref[...], 0.0)


def _input_proj(atom_p, w_p, b):
    return pl.pallas_call(
        _proj_body,
        grid=(10,),
        in_specs=[
            pl.BlockSpec((_N // 10, _DINP), lambda i: (i, 0)),
            pl.BlockSpec((_DINP, _H), lambda i: (0, 0)),
            pl.BlockSpec((1, _H), lambda i: (0, 0)),
        ],
        out_specs=pl.BlockSpec((_N // 10, _H), lambda i: (i, 0)),
        out_shape=jax.ShapeDtypeStruct((_N, _H), jnp.float32),
    )(atom_p, w_p, b)


def _bw_body(bond_ref, w_ref, b_ref, o_ref):
    o_ref[...] = jnp.dot(bond_ref[...], w_ref[...],
                         preferred_element_type=jnp.float32) + b_ref[...]


def _bond_proj(bond_p, we, b):
    blk = 4096
    return pl.pallas_call(
        _bw_body,
        grid=(_EPAD // blk,),
        in_specs=[
            pl.BlockSpec((blk, _DE), lambda i: (i, 0)),
            pl.BlockSpec((_DE, _H), lambda i: (0, 0)),
            pl.BlockSpec((1, _H), lambda i: (0, 0)),
        ],
        out_specs=pl.BlockSpec((blk, _H), lambda i: (i, 0)),
        out_shape=jax.ShapeDtypeStruct((_EPAD, _H), jnp.float32),
    )(bond_p, we, b)


def _hw_body(h_ref, w_ref, o_ref):
    o_ref[...] = jnp.dot(h_ref[...], w_ref[...],
                         preferred_element_type=jnp.float32)


def _node_msg(h, wh):
    return pl.pallas_call(
        _hw_body,
        out_shape=jax.ShapeDtypeStruct((_N, _H), jnp.float32),
    )(h, wh)


def _upd_body(a0_ref, a1_ref, h_ref, w_ref, b_ref, o_ref):
    agg = a0_ref[...] + a1_ref[...]
    o_ref[...] = jnp.maximum(
        jnp.dot(agg, w_ref[...], preferred_element_type=jnp.float32)
        + b_ref[...], 0.0) + h_ref[...]


def _update(a0, a1, h, w, b):
    return pl.pallas_call(
        _upd_body,
        out_shape=jax.ShapeDtypeStruct((_N, _H), jnp.float32),
    )(a0, a1, h, w, b)


def _head_body(h_ref, w_ref, b_ref, o_ref):
    s = jnp.sum(h_ref[...] * w_ref[...], axis=1, keepdims=True) + b_ref[...]
    o_ref[...] = jax.nn.sigmoid(s)


def _head(h, w_row, b):
    return pl.pallas_call(
        _head_body,
        out_shape=jax.ShapeDtypeStruct((_N, 1), jnp.float32),
    )(h, w_row, b)


_MESH = plsc.VectorSubcoreMesh(core_axis_name="c", subcore_axis_name="s")


@functools.partial(
    pl.kernel,
    out_type=jax.ShapeDtypeStruct((_NC, _NPAD, _H), jnp.float32),
    mesh=_MESH,
    scratch_types=[
        pltpu.VMEM((_CH,), jnp.int32),
        pltpu.VMEM((_CH,), jnp.int32),
        pltpu.VMEM((_CH, _H), jnp.float32),
        pltpu.VMEM((_CH, _H), jnp.float32),
        pltpu.VMEM_SHARED((_NPAD, _H), jnp.float32),
        pltpu.SemaphoreType.DMA,
    ],
)
def _edge_kernel(hw_hbm, bw_hbm, src_hbm, dst_hbm, out_hbm,
                 src_v, dst_v, rows_v, bw_v, agg_sh, sem):
    c = lax.axis_index("c")
    s = lax.axis_index("s")
    wid = c * _NS + s

    zero16 = jnp.zeros((16,), jnp.float32)

    def zbody(i, _):
        for j in range(_H // 16):
            bw_v[i, pl.ds(j * 16, 16)] = zero16
        return ()

    lax.fori_loop(0, _CH, zbody, (), unroll=False)
    for r in range(_RPT // _CH):
        pltpu.sync_copy(bw_v, agg_sh.at[pl.ds(s * _RPT + r * _CH, _CH)])
    plsc.subcore_barrier()

    def chunk_body(k, _):
        base = pl.multiple_of(wid * _EPW + k * _CH, _CH)
        pltpu.sync_copy(src_hbm.at[pl.ds(base, _CH)], src_v)
        pltpu.sync_copy(dst_hbm.at[pl.ds(base, _CH)], dst_v)
        pltpu.async_copy(hw_hbm.at[src_v], rows_v, sem).wait()
        pltpu.sync_copy(bw_hbm.at[pl.ds(base, _CH)], bw_v)

        def ebody(e, _):
            for j in range(_H // 16):
                sl = pl.ds(j * 16, 16)
                rows_v[e, sl] = jnp.maximum(rows_v[e, sl] + bw_v[e, sl], 0.0)
            return ()

        lax.fori_loop(0, _CH, ebody, (), unroll=False)
        pltpu.sync_copy(rows_v, agg_sh.at[dst_v], add=True)
        return ()

    lax.fori_loop(0, _NCHUNK, chunk_body, (), unroll=False)
    plsc.subcore_barrier()

    pltpu.sync_copy(agg_sh.at[pl.ds(s * _RPT, _RPT)],
                    out_hbm.at[c, pl.ds(s * _RPT, _RPT)])


def kernel(atom_feature, edge_index, bond_feature, node2graph,
           W_in, b_in, W_msg, b_msg, W_upd, b_upd, W_out, b_out):
    src = edge_index[0]
    dst = edge_index[1]
    padn = _EPAD - _E
    fill = jnp.arange(padn, dtype=jnp.int32)
    src_p = jnp.concatenate([src, (fill * 997) % _N])
    dst_p = jnp.concatenate([dst, _N + fill % (_NPAD - _N)])
    bond_p = jnp.pad(bond_feature, ((0, padn), (0, 0)))
    atom_p = jnp.pad(atom_feature, ((0, 0), (0, _DINP - _DIN)))
    w_in_p = jnp.pad(W_in, ((0, _DINP - _DIN), (0, 0)))

    h = _input_proj(atom_p, w_in_p, b_in.reshape(1, _H))
    wh = W_msg[:, :_H, :]
    we = W_msg[:, _H:, :]
    for i in range(_NBLK):
        bw = _bond_proj(bond_p, we[i], b_msg[i].reshape(1, _H))
        hw = _node_msg(h, wh[i])
        parts = _edge_kernel(hw, bw, src_p, dst_p)
        h = _update(parts[0, :_N], parts[1, :_N], h, W_upd[i],
                    b_upd[i].reshape(1, _H))
    out = _head(h, W_out.reshape(1, _H), b_out.reshape(1, 1))
    return out[:, 0]


# speedup vs baseline: 4.7285x; 1.6151x over previous
import functools

import jax
import jax.numpy as jnp
from jax import lax
from jax.experimental import pallas as pl
from jax.experimental.pallas import tpu as pltpu
from jax.experimental.pallas import tpu_sc as plsc

_N, _E, _DIN, _H, _DE, _NBLK = 10000, 320000, 1422, 128, 8, 5
_NC, _NS = 2, 16
_NW = _NC * _NS
_CH = 88
_NCHUNK = 114
_EPW = _NCHUNK * _CH
_EPAD = _NW * _EPW
_EPAD2 = 323584
_NPAD = 10112
_RPT = _NPAD // _NS
_DINP = 1424


def _proj_body(a_ref, w_ref, b_ref, o_ref):
    o_ref[...] = jnp.maximum(
        jnp.dot(a_ref[...], w_ref[...], preferred_element_type=jnp.float32)
        + b_ref[...], 0.0)


def _input_proj(atom_p, w_p, b):
    return pl.pallas_call(
        _proj_body,
        grid=(10,),
        in_specs=[
            pl.BlockSpec((_N // 10, _DINP), lambda i: (i, 0)),
            pl.BlockSpec((_DINP, _H), lambda i: (0, 0)),
            pl.BlockSpec((1, _H), lambda i: (0, 0)),
        ],
        out_specs=pl.BlockSpec((_N // 10, _H), lambda i: (i, 0)),
        out_shape=jax.ShapeDtypeStruct((_N, _H), jnp.float32),
    )(atom_p, w_p, b)


def _bw_body(bond_ref, w_ref, b_ref, o_ref):
    o_ref[...] = jnp.dot(bond_ref[...], w_ref[...],
                         preferred_element_type=jnp.float32) + b_ref[...]


def _bond_proj(bond_p, we, b):
    blk = 4096
    return pl.pallas_call(
        _bw_body,
        grid=(_EPAD2 // blk,),
        in_specs=[
            pl.BlockSpec((blk, _DE), lambda i: (i, 0)),
            pl.BlockSpec((_DE, _H), lambda i: (0, 0)),
            pl.BlockSpec((1, _H), lambda i: (0, 0)),
        ],
        out_specs=pl.BlockSpec((blk, _H), lambda i: (i, 0)),
        out_shape=jax.ShapeDtypeStruct((_EPAD2, _H), jnp.float32),
    )(bond_p, we, b)


def _hw_body(h_ref, w_ref, o_ref):
    o_ref[...] = jnp.dot(h_ref[...], w_ref[...],
                         preferred_element_type=jnp.float32)


def _node_msg(h, wh):
    return pl.pallas_call(
        _hw_body,
        out_shape=jax.ShapeDtypeStruct((_N, _H), jnp.float32),
    )(h, wh)


def _upd_body(a0_ref, a1_ref, h_ref, w_ref, b_ref, o_ref):
    agg = a0_ref[...] + a1_ref[...]
    o_ref[...] = jnp.maximum(
        jnp.dot(agg, w_ref[...], preferred_element_type=jnp.float32)
        + b_ref[...], 0.0) + h_ref[...]


def _update(a0, a1, h, w, b):
    return pl.pallas_call(
        _upd_body,
        out_shape=jax.ShapeDtypeStruct((_N, _H), jnp.float32),
    )(a0, a1, h, w, b)


def _head_body(h_ref, w_ref, b_ref, o_ref):
    s = jnp.sum(h_ref[...] * w_ref[...], axis=1, keepdims=True) + b_ref[...]
    o_ref[...] = jax.nn.sigmoid(s)


def _head(h, w_row, b):
    return pl.pallas_call(
        _head_body,
        out_shape=jax.ShapeDtypeStruct((_N, 1), jnp.float32),
    )(h, w_row, b)


_MESH = plsc.VectorSubcoreMesh(core_axis_name="c", subcore_axis_name="s")


@functools.partial(
    pl.kernel,
    out_type=jax.ShapeDtypeStruct((_NC, _NPAD, _H), jnp.float32),
    mesh=_MESH,
    scratch_types=[
        pltpu.VMEM((2, _CH), jnp.int32),
        pltpu.VMEM((2, _CH), jnp.int32),
        pltpu.VMEM((_CH, _H), jnp.float32),
        pltpu.VMEM((_CH, _H), jnp.float32),
        pltpu.VMEM((_CH, _H), jnp.float32),
        pltpu.VMEM((_CH, _H), jnp.float32),
        pltpu.VMEM_SHARED((_NPAD, _H), jnp.float32),
        pltpu.SemaphoreType.DMA,
        pltpu.SemaphoreType.DMA,
        pltpu.SemaphoreType.DMA,
        pltpu.SemaphoreType.DMA,
        pltpu.SemaphoreType.DMA,
        pltpu.SemaphoreType.DMA,
        pltpu.SemaphoreType.DMA,
        pltpu.SemaphoreType.DMA,
    ],
)
def _edge_kernel(hw_hbm, bw_hbm, src_hbm, dst_hbm, out_hbm,
                 srcb_v, dstb_v, rows0_v, rows1_v, bwb0_v, bwb1_v, agg_sh,
                 ss0, ss1, sd0, sd1, g0, g1, w0, w1):
    rows_b = (rows0_v, rows1_v)
    bwb_b = (bwb0_v, bwb1_v)
    ss_b = (ss0, ss1)
    sd_b = (sd0, sd1)
    g_b = (g0, g1)
    w_b = (w0, w1)

    c = lax.axis_index("c")
    s = lax.axis_index("s")
    wid = c * _NS + s
    ebase = pl.multiple_of(wid * _EPW, 8)

    zero16 = jnp.zeros((16,), jnp.float32)

    def zbody(i, _):
        for j in range(_H // 16):
            rows0_v[i, pl.ds(j * 16, 16)] = zero16
        return ()

    lax.fori_loop(0, _CH, zbody, (), unroll=False)
    for r in range(7):
        pltpu.sync_copy(rows0_v,
                        agg_sh.at[pl.ds(s * _RPT + r * _CH, _CH)])
    pltpu.sync_copy(rows0_v.at[pl.ds(0, _RPT - 7 * _CH)],
                    agg_sh.at[pl.ds(s * _RPT + 7 * _CH, _RPT - 7 * _CH)])
    plsc.subcore_barrier()

    def start_src(k, b):
        pltpu.async_copy(src_hbm.at[pl.ds(ebase + k * _CH, _CH)],
                         srcb_v.at[b], ss_b[b])

    def start_dst(k, b):
        pltpu.async_copy(dst_hbm.at[pl.ds(ebase + k * _CH, _CH)],
                         dstb_v.at[b], sd_b[b])

    def wait_src(b):
        pltpu.make_async_copy(src_hbm.at[pl.ds(0, _CH)], srcb_v.at[b],
                              ss_b[b]).wait()

    def wait_dst(b):
        pltpu.make_async_copy(dst_hbm.at[pl.ds(0, _CH)], dstb_v.at[b],
                              sd_b[b]).wait()

    def start_fetch(k, b):
        pltpu.async_copy(hw_hbm.at[srcb_v.at[b]], rows_b[b], g_b[b])
        pltpu.async_copy(bw_hbm.at[pl.ds(ebase + k * _CH, _CH)],
                         bwb_b[b], w_b[b])

    def wait_fetch(b):
        pltpu.make_async_copy(hw_hbm.at[srcb_v.at[b]], rows_b[b],
                              g_b[b]).wait()
        pltpu.make_async_copy(bw_hbm.at[pl.ds(0, _CH)], bwb_b[b],
                              w_b[b]).wait()

    def compute(b):
        rv, bv = rows_b[b], bwb_b[b]

        def ebody(e, _):
            for j in range(_H // 16):
                sl = pl.ds(j * 16, 16)
                rv[e, sl] = jnp.maximum(rv[e, sl] + bv[e, sl], 0.0)
            return ()

        lax.fori_loop(0, _CH, ebody, (), unroll=False)

    def scatter(b):
        pltpu.sync_copy(rows_b[b], agg_sh.at[dstb_v.at[b]], add=True)

    start_src(0, 0)
    start_dst(0, 0)
    wait_src(0)
    start_fetch(0, 0)
    start_src(1, 1)
    start_dst(1, 1)

    def loop_body(k2, _):
        for b in range(2):
            k = k2 * 2 + b
            nb = 1 - b

            wait_fetch(b)

            @pl.when(k + 2 < _NCHUNK)
            def _():
                start_src(k + 2, b)

            @pl.when(k + 1 < _NCHUNK)
            def _():
                wait_src(nb)
                start_fetch(k + 1, nb)

            compute(b)
            wait_dst(b)
            scatter(b)

            @pl.when(k + 2 < _NCHUNK)
            def _():
                start_dst(k + 2, b)
        return ()

    lax.fori_loop(0, _NCHUNK // 2, loop_body, (), unroll=False)
    plsc.subcore_barrier()

    pltpu.sync_copy(agg_sh.at[pl.ds(s * _RPT, _RPT)],
                    out_hbm.at[c, pl.ds(s * _RPT, _RPT)])


def kernel(atom_feature, edge_index, bond_feature, node2graph,
           W_in, b_in, W_msg, b_msg, W_upd, b_upd, W_out, b_out):
    src = edge_index[0]
    dst = edge_index[1]
    padn = _EPAD - _E
    fill = jnp.arange(padn, dtype=jnp.int32)
    src_p = jnp.concatenate([src, (fill * 997) % _N])
    dst_p = jnp.concatenate([dst, _N + fill % (_NPAD - _N)])
    bond_p = jnp.pad(bond_feature, ((0, _EPAD2 - _E), (0, 0)))
    atom_p = jnp.pad(atom_feature, ((0, 0), (0, _DINP - _DIN)))
    w_in_p = jnp.pad(W_in, ((0, _DINP - _DIN), (0, 0)))

    h = _input_proj(atom_p, w_in_p, b_in.reshape(1, _H))
    wh = W_msg[:, :_H, :]
    we = W_msg[:, _H:, :]
    for i in range(_NBLK):
        bw = _bond_proj(bond_p, we[i], b_msg[i].reshape(1, _H))
        hw = _node_msg(h, wh[i])
        parts = _edge_kernel(hw, bw, src_p, dst_p)
        h = _update(parts[0, :_N], parts[1, :_N], h, W_upd[i],
                    b_upd[i].reshape(1, _H))
    out = _head(h, W_out.reshape(1, _H), b_out.reshape(1, 1))
    return out[:, 0]


# speedup vs baseline: 4.9776x; 1.0527x over previous
import functools

import jax
import jax.numpy as jnp
from jax import lax
from jax.experimental import pallas as pl
from jax.experimental.pallas import tpu as pltpu
from jax.experimental.pallas import tpu_sc as plsc

_N, _E, _DIN, _H, _DE, _NBLK = 10000, 320000, 1422, 128, 8, 5
_NC, _NS = 2, 16
_NW = _NC * _NS
_CH = 88
_NCHUNK = 114
_EPW = _NCHUNK * _CH
_EPAD = _NW * _EPW
_EPAD2 = 323584
_NPAD = 10112
_RPT = _NPAD // _NS


def _proj_body(a_ref, w_ref, b_ref, o_ref):
    o_ref[...] = jnp.maximum(
        jnp.dot(a_ref[...], w_ref[...], preferred_element_type=jnp.float32)
        + b_ref[...], 0.0)


def _input_proj(atom, w, b):
    return pl.pallas_call(
        _proj_body,
        grid=(10,),
        in_specs=[
            pl.BlockSpec((_N // 10, _DIN), lambda i: (i, 0)),
            pl.BlockSpec((_DIN, _H), lambda i: (0, 0)),
            pl.BlockSpec((1, _H), lambda i: (0, 0)),
        ],
        out_specs=pl.BlockSpec((_N // 10, _H), lambda i: (i, 0)),
        out_shape=jax.ShapeDtypeStruct((_N, _H), jnp.float32),
    )(atom, w, b)


def _bw_body(bond_ref, w_ref, b_ref, o_ref):
    o_ref[...] = jnp.dot(bond_ref[...], w_ref[...],
                         preferred_element_type=jnp.float32) + b_ref[...]


def _bond_proj(bond_p, we, b):
    blk = 4096
    return pl.pallas_call(
        _bw_body,
        grid=(_EPAD2 // blk,),
        in_specs=[
            pl.BlockSpec((blk, _DE), lambda i: (i, 0)),
            pl.BlockSpec((_DE, _H), lambda i: (0, 0)),
            pl.BlockSpec((1, _H), lambda i: (0, 0)),
        ],
        out_specs=pl.BlockSpec((blk, _H), lambda i: (i, 0)),
        out_shape=jax.ShapeDtypeStruct((_EPAD2, _H), jnp.float32),
    )(bond_p, we, b)


def _hw_body(h_ref, w_ref, o_ref):
    o_ref[...] = jnp.dot(h_ref[...], w_ref[...],
                         preferred_element_type=jnp.float32)


def _node_msg(h, wh):
    return pl.pallas_call(
        _hw_body,
        out_shape=jax.ShapeDtypeStruct((_N, _H), jnp.float32),
    )(h, wh)


def _upd_body(parts_ref, h_ref, w_ref, b_ref, o_ref):
    agg = parts_ref[0] + parts_ref[1]
    o_ref[...] = jnp.maximum(
        jnp.dot(agg, w_ref[...], preferred_element_type=jnp.float32)
        + b_ref[...], 0.0) + h_ref[...]


def _update(parts, h, w, b):
    return pl.pallas_call(
        _upd_body,
        grid=(1,),
        in_specs=[
            pl.BlockSpec((_NC, _N, _H), lambda i: (0, 0, 0)),
            pl.BlockSpec((_N, _H), lambda i: (0, 0)),
            pl.BlockSpec((_H, _H), lambda i: (0, 0)),
            pl.BlockSpec((1, _H), lambda i: (0, 0)),
        ],
        out_specs=pl.BlockSpec((_N, _H), lambda i: (0, 0)),
        out_shape=jax.ShapeDtypeStruct((_N, _H), jnp.float32),
    )(parts, h, w, b)


def _head_body(h_ref, w_ref, b_ref, o_ref):
    s = jnp.sum(h_ref[...] * w_ref[...], axis=1, keepdims=True) + b_ref[...]
    o_ref[...] = jax.nn.sigmoid(s)


def _head(h, w_row, b):
    return pl.pallas_call(
        _head_body,
        out_shape=jax.ShapeDtypeStruct((_N, 1), jnp.float32),
    )(h, w_row, b)


_MESH = plsc.VectorSubcoreMesh(core_axis_name="c", subcore_axis_name="s")


@functools.partial(
    pl.kernel,
    out_type=jax.ShapeDtypeStruct((_NC, _NPAD, _H), jnp.float32),
    mesh=_MESH,
    scratch_types=[
        pltpu.VMEM((2, _CH), jnp.int32),
        pltpu.VMEM((2, _CH), jnp.int32),
        pltpu.VMEM((_CH, _H), jnp.float32),
        pltpu.VMEM((_CH, _H), jnp.float32),
        pltpu.VMEM((_CH, _H), jnp.float32),
        pltpu.VMEM((_CH, _H), jnp.float32),
        pltpu.VMEM_SHARED((_NPAD, _H), jnp.float32),
        pltpu.SemaphoreType.DMA,
        pltpu.SemaphoreType.DMA,
        pltpu.SemaphoreType.DMA,
        pltpu.SemaphoreType.DMA,
        pltpu.SemaphoreType.DMA,
        pltpu.SemaphoreType.DMA,
        pltpu.SemaphoreType.DMA,
        pltpu.SemaphoreType.DMA,
    ],
)
def _edge_kernel(hw_hbm, bw_hbm, src_hbm, dst_hbm, out_hbm,
                 srcb_v, dstb_v, rows0_v, rows1_v, bwb0_v, bwb1_v, agg_sh,
                 ss0, ss1, sd0, sd1, g0, g1, w0, w1):
    rows_b = (rows0_v, rows1_v)
    bwb_b = (bwb0_v, bwb1_v)
    ss_b = (ss0, ss1)
    sd_b = (sd0, sd1)
    g_b = (g0, g1)
    w_b = (w0, w1)

    c = lax.axis_index("c")
    s = lax.axis_index("s")
    wid = c * _NS + s
    ebase = pl.multiple_of(wid * _EPW, 8)

    zero16 = jnp.zeros((16,), jnp.float32)

    def zbody(i, _):
        for j in range(_H // 16):
            rows0_v[i, pl.ds(j * 16, 16)] = zero16
        return ()

    lax.fori_loop(0, _CH, zbody, (), unroll=False)

    for r in range(7):
        pltpu.sync_copy(rows0_v,
                        agg_sh.at[pl.ds(s * _RPT + r * _CH, _CH)])
    pltpu.sync_copy(rows0_v.at[pl.ds(0, _RPT - 7 * _CH)],
                    agg_sh.at[pl.ds(s * _RPT + 7 * _CH, _RPT - 7 * _CH)])
    plsc.subcore_barrier()

    def start_src(k, b):
        pltpu.async_copy(src_hbm.at[pl.ds(ebase + k * _CH, _CH)],
                         srcb_v.at[b], ss_b[b])

    def start_dst(k, b):
        pltpu.async_copy(dst_hbm.at[pl.ds(ebase + k * _CH, _CH)],
                         dstb_v.at[b], sd_b[b])

    def wait_src(b):
        pltpu.make_async_copy(src_hbm.at[pl.ds(0, _CH)], srcb_v.at[b],
                              ss_b[b]).wait()

    def wait_dst(b):
        pltpu.make_async_copy(dst_hbm.at[pl.ds(0, _CH)], dstb_v.at[b],
                              sd_b[b]).wait()

    def start_fetch(k, b):
        pltpu.async_copy(hw_hbm.at[srcb_v.at[b]], rows_b[b], g_b[b])
        pltpu.async_copy(bw_hbm.at[pl.ds(ebase + k * _CH, _CH)],
                         bwb_b[b], w_b[b])

    def wait_fetch(b):
        pltpu.make_async_copy(hw_hbm.at[srcb_v.at[b]], rows_b[b],
                              g_b[b]).wait()
        pltpu.make_async_copy(bw_hbm.at[pl.ds(0, _CH)], bwb_b[b],
                              w_b[b]).wait()

    def compute(b):
        rv, bv = rows_b[b], bwb_b[b]

        def ebody(e, _):
            for j in range(_H // 16):
                sl = pl.ds(j * 16, 16)
                rv[e, sl] = jnp.maximum(rv[e, sl] + bv[e, sl], 0.0)
            return ()

        lax.fori_loop(0, _CH, ebody, (), unroll=False)

    def scatter(b):
        pltpu.sync_copy(rows_b[b], agg_sh.at[dstb_v.at[b]], add=True)

    start_src(0, 0)
    start_dst(0, 0)
    wait_src(0)
    start_fetch(0, 0)
    start_src(1, 1)
    start_dst(1, 1)

    def loop_body(k2, _):
        for b in range(2):
            k = k2 * 2 + b
            nb = 1 - b

            wait_fetch(b)

            @pl.when(k + 2 < _NCHUNK)
            def _():
                start_src(k + 2, b)

            @pl.when(k + 1 < _NCHUNK)
            def _():
                wait_src(nb)
                start_fetch(k + 1, nb)

            compute(b)
            wait_dst(b)
            scatter(b)

            @pl.when(k + 2 < _NCHUNK)
            def _():
                start_dst(k + 2, b)
        return ()

    lax.fori_loop(0, _NCHUNK // 2, loop_body, (), unroll=False)
    plsc.subcore_barrier()

    pltpu.sync_copy(agg_sh.at[pl.ds(s * _RPT, _RPT)],
                    out_hbm.at[c, pl.ds(s * _RPT, _RPT)])


def kernel(atom_feature, edge_index, bond_feature, node2graph,
           W_in, b_in, W_msg, b_msg, W_upd, b_upd, W_out, b_out):
    src = edge_index[0]
    dst = edge_index[1]
    padn = _EPAD - _E
    fill = jnp.arange(padn, dtype=jnp.int32)
    src_p = jnp.concatenate([src, (fill * 997) % _N])
    dst_p = jnp.concatenate([dst, _N + fill % (_NPAD - _N)])
    bond_p = jnp.pad(bond_feature, ((0, _EPAD2 - _E), (0, 0)))

    h = _input_proj(atom_feature, W_in, b_in.reshape(1, _H))
    wh = W_msg[:, :_H, :]
    we = W_msg[:, _H:, :]
    for i in range(_NBLK):
        bw = _bond_proj(bond_p, we[i], b_msg[i].reshape(1, _H))
        hw = _node_msg(h, wh[i])
        parts = _edge_kernel(hw, bw, src_p, dst_p)
        h = _update(parts, h, W_upd[i], b_upd[i].reshape(1, _H))
    out = _head(h, W_out.reshape(1, _H), b_out.reshape(1, 1))
    return out[:, 0]
